# Initial kernel scaffold; baseline (speedup 1.0000x reference)
import jax
import jax.numpy as jnp
from jax.experimental import pallas as pl


def kernel(head, relation, tail, entity_emb, relation_emb):
    raise NotImplementedError("write your pallas kernel here")


# speedup vs baseline: 1.2358x; 1.2358x over previous
import functools

import jax
import jax.numpy as jnp
from jax import lax
from jax.experimental import pallas as pl
from jax.experimental.pallas import tpu as pltpu
from jax.experimental.pallas import tpu_sc as plsc

NC = 2
NS = 16
L = 16
NW = NC * NS


def _fast_sqrt(x):
    x = jnp.maximum(x, jnp.float32(1e-30))
    i = lax.bitcast_convert_type(x, jnp.int32)
    i = jnp.int32(0x5F3759DF) - lax.shift_right_arithmetic(i, jnp.int32(1))
    y = lax.bitcast_convert_type(i, jnp.float32)
    y = y * (jnp.float32(1.5) - jnp.float32(0.5) * x * y * y)
    y = y * (jnp.float32(1.5) - jnp.float32(0.5) * x * y * y)
    return x * y


@functools.lru_cache(maxsize=None)
def _build_sc_kernel(B, D):
    BPW = B // NW
    CH = 128
    NCH = BPW // CH
    KD = D // L

    mesh = plsc.VectorSubcoreMesh(core_axis_name="c", subcore_axis_name="s")

    @functools.partial(
        pl.kernel,
        mesh=mesh,
        compiler_params=pltpu.CompilerParams(needs_layout_passes=False),
        out_type=jax.ShapeDtypeStruct((NW, BPW), jnp.float32),
        scratch_types=[
            pltpu.VMEM((NCH, CH), jnp.int32),
            pltpu.VMEM((NCH, CH), jnp.int32),
            pltpu.VMEM((NCH, CH), jnp.int32),
            pltpu.VMEM((CH, D), jnp.float32),
            pltpu.VMEM((CH, D), jnp.float32),
            pltpu.VMEM((CH, D), jnp.float32),
            pltpu.VMEM((BPW,), jnp.float32),
            pltpu.SemaphoreType.DMA,
            pltpu.SemaphoreType.DMA,
            pltpu.SemaphoreType.DMA,
        ],
    )
    def sc_kernel(head_h, rel_h, tail_h, ent_h, remb_h, out_h,
                  idx_h, idx_r, idx_t, hb, rb, tb, outb, sh, sr, st):
        wid = lax.axis_index("s") * NC + lax.axis_index("c")
        pltpu.sync_copy(head_h.at[wid], idx_h)
        pltpu.sync_copy(rel_h.at[wid], idx_r)
        pltpu.sync_copy(tail_h.at[wid], idx_t)

        rows16 = lax.iota(jnp.int32, L)

        for c in range(NCH):
            cp_h = pltpu.async_copy(ent_h.at[idx_h.at[c]], hb, sh)
            cp_r = pltpu.async_copy(remb_h.at[idx_r.at[c]], rb, sr)
            cp_t = pltpu.async_copy(ent_h.at[idx_t.at[c]], tb, st)
            cp_h.wait()
            cp_r.wait()
            cp_t.wait()

            def group(g, _, c=c):
                tot = jnp.zeros((L,), jnp.float32)
                for j in range(L):
                    row = g * L + j
                    acc = jnp.zeros((L,), jnp.float32)
                    for k in range(KD):
                        h = hb[row, pl.ds(k * L, L)]
                        r = rb[row, pl.ds(k * L, L)]
                        t = tb[row, pl.ds(k * L, L)]
                        d = h + r - t
                        acc = acc + d * d
                    s = jnp.sum(acc)
                    tot = jnp.where(rows16 == j, s, tot)
                outb[pl.ds(c * CH + g * L, L)] = _fast_sqrt(tot)
                return 0

            lax.fori_loop(0, CH // L, group, 0)

        pltpu.sync_copy(outb, out_h.at[wid])

    return sc_kernel


def kernel(head, relation, tail, entity_emb, relation_emb):
    B = head.shape[0]
    D = entity_emb.shape[1]
    BPW = B // NW
    CH = 128
    NCH = BPW // CH
    sc_kernel = _build_sc_kernel(B, D)
    score = sc_kernel(
        head.reshape(NW, NCH, CH),
        relation.reshape(NW, NCH, CH),
        tail.reshape(NW, NCH, CH),
        entity_emb,
        relation_emb,
    )
    return score.reshape(B)


# speedup vs baseline: 1.4445x; 1.1689x over previous
import functools

import jax
import jax.numpy as jnp
from jax import lax
from jax.experimental import pallas as pl
from jax.experimental.pallas import tpu as pltpu
from jax.experimental.pallas import tpu_sc as plsc

NC = 2
NS = 16
L = 16
NW = NC * NS


def _fast_sqrt(x):
    x = jnp.maximum(x, jnp.float32(1e-30))
    i = lax.bitcast_convert_type(x, jnp.int32)
    i = jnp.int32(0x5F3759DF) - lax.shift_right_arithmetic(i, jnp.int32(1))
    y = lax.bitcast_convert_type(i, jnp.float32)
    y = y * (jnp.float32(1.5) - jnp.float32(0.5) * x * y * y)
    y = y * (jnp.float32(1.5) - jnp.float32(0.5) * x * y * y)
    return x * y


@functools.lru_cache(maxsize=None)
def _build_sc_kernel(B, D):
    BPW = B // NW
    CH = 128
    NCH = BPW // CH
    KD = D // L

    mesh = plsc.VectorSubcoreMesh(core_axis_name="c", subcore_axis_name="s")

    @functools.partial(
        pl.kernel,
        mesh=mesh,
        compiler_params=pltpu.CompilerParams(needs_layout_passes=False),
        out_type=jax.ShapeDtypeStruct((NW, BPW), jnp.float32),
        scratch_types=[
            pltpu.VMEM((NCH, CH), jnp.int32),
            pltpu.VMEM((NCH, CH), jnp.int32),
            pltpu.VMEM((NCH, CH), jnp.int32),
            pltpu.VMEM((2, CH, D), jnp.float32),
            pltpu.VMEM((2, CH, D), jnp.float32),
            pltpu.VMEM((2, CH, D), jnp.float32),
            pltpu.VMEM((BPW,), jnp.float32),
            pltpu.SemaphoreType.DMA,
            pltpu.SemaphoreType.DMA,
            pltpu.SemaphoreType.DMA,
            pltpu.SemaphoreType.DMA,
            pltpu.SemaphoreType.DMA,
            pltpu.SemaphoreType.DMA,
        ],
    )
    def sc_kernel(head_h, rel_h, tail_h, ent_h, remb_h, out_h,
                  idx_h, idx_r, idx_t, hb, rb, tb, outb,
                  sh0, sh1, sr0, sr1, st0, st1):
        wid = lax.axis_index("s") * NC + lax.axis_index("c")
        pltpu.sync_copy(head_h.at[wid], idx_h)
        pltpu.sync_copy(rel_h.at[wid], idx_r)
        pltpu.sync_copy(tail_h.at[wid], idx_t)

        sems = ((sh0, sr0, st0), (sh1, sr1, st1))
        rows16 = lax.iota(jnp.int32, L)

        def issue(c):
            slot = c % 2
            sh, sr, st = sems[slot]
            return (
                pltpu.async_copy(ent_h.at[idx_h.at[c]], hb.at[slot], sh),
                pltpu.async_copy(remb_h.at[idx_r.at[c]], rb.at[slot], sr),
                pltpu.async_copy(ent_h.at[idx_t.at[c]], tb.at[slot], st),
            )

        inflight = [None, None]
        inflight[0] = issue(0)
        for c in range(NCH):
            if c + 1 < NCH:
                inflight[(c + 1) % 2] = issue(c + 1)
            slot = c % 2
            for cp in inflight[slot]:
                cp.wait()

            def group(g, _, c=c, slot=slot):
                tot = jnp.zeros((L,), jnp.float32)
                for j in range(L):
                    row = g * L + j
                    acc0 = jnp.zeros((L,), jnp.float32)
                    acc1 = jnp.zeros((L,), jnp.float32)
                    for k in range(KD):
                        h = hb[slot, row, pl.ds(k * L, L)]
                        r = rb[slot, row, pl.ds(k * L, L)]
                        t = tb[slot, row, pl.ds(k * L, L)]
                        d = h + r - t
                        if k % 2 == 0:
                            acc0 = acc0 + d * d
                        else:
                            acc1 = acc1 + d * d
                    s = jnp.sum(acc0 + acc1)
                    tot = jnp.where(rows16 == j, s, tot)
                outb[pl.ds(c * CH + g * L, L)] = _fast_sqrt(tot)
                return 0

            lax.fori_loop(0, CH // L, group, 0)

        pltpu.sync_copy(outb, out_h.at[wid])

    return sc_kernel


def kernel(head, relation, tail, entity_emb, relation_emb):
    B = head.shape[0]
    D = entity_emb.shape[1]
    BPW = B // NW
    CH = 128
    NCH = BPW // CH
    sc_kernel = _build_sc_kernel(B, D)
    score = sc_kernel(
        head.reshape(NW, NCH, CH),
        relation.reshape(NW, NCH, CH),
        tail.reshape(NW, NCH, CH),
        entity_emb,
        relation_emb,
    )
    return score.reshape(B)


# speedup vs baseline: 2.5378x; 1.7568x over previous
import functools

import jax
import jax.numpy as jnp
from jax import lax
from jax.experimental import pallas as pl
from jax.experimental.pallas import tpu as pltpu
from jax.experimental.pallas import tpu_sc as plsc

NC = 2
NS = 16
L = 16
NW = NC * NS


def _fast_sqrt(x):
    x = jnp.maximum(x, jnp.float32(1e-30))
    i = lax.bitcast_convert_type(x, jnp.int32)
    i = jnp.int32(0x5F3759DF) - lax.shift_right_arithmetic(i, jnp.int32(1))
    y = lax.bitcast_convert_type(i, jnp.float32)
    y = y * (jnp.float32(1.5) - jnp.float32(0.5) * x * y * y)
    y = y * (jnp.float32(1.5) - jnp.float32(0.5) * x * y * y)
    return x * y


@functools.lru_cache(maxsize=None)
def _build_sc_kernel(B, D):
    BPW = B // NW
    CH = 128
    NCH = BPW // CH
    KD = D // L

    mesh = plsc.VectorSubcoreMesh(core_axis_name="c", subcore_axis_name="s")

    @functools.partial(
        pl.kernel,
        mesh=mesh,
        compiler_params=pltpu.CompilerParams(needs_layout_passes=False),
        out_type=jax.ShapeDtypeStruct((NW, BPW), jnp.float32),
        scratch_types=[
            pltpu.VMEM((3, NCH, CH), jnp.int32),
            pltpu.VMEM((2, CH, D), jnp.float32),
            pltpu.VMEM((2, CH, D), jnp.float32),
            pltpu.VMEM((2, CH, D), jnp.float32),
            pltpu.VMEM((BPW,), jnp.float32),
            pltpu.SemaphoreType.DMA,
            pltpu.SemaphoreType.DMA,
            pltpu.SemaphoreType.DMA,
            pltpu.SemaphoreType.DMA,
            pltpu.SemaphoreType.DMA,
            pltpu.SemaphoreType.DMA,
        ],
    )
    def sc_kernel(hrt_h, ent_h, remb_h, out_h,
                  idx, hb, rb, tb, outb,
                  sh0, sh1, sr0, sr1, st0, st1):
        wid = lax.axis_index("s") * NC + lax.axis_index("c")
        pltpu.sync_copy(hrt_h.at[wid], idx)

        sems = ((sh0, sr0, st0), (sh1, sr1, st1))
        lane = lax.iota(jnp.int32, L)

        def issue(c):
            slot = c % 2
            sh, sr, st = sems[slot]
            return (
                pltpu.async_copy(ent_h.at[idx.at[0, c]], hb.at[slot], sh),
                pltpu.async_copy(remb_h.at[idx.at[1, c]], rb.at[slot], sr),
                pltpu.async_copy(ent_h.at[idx.at[2, c]], tb.at[slot], st),
            )

        inflight = [None, None]
        inflight[0] = issue(0)
        for c in range(NCH):
            if c + 1 < NCH:
                inflight[(c + 1) % 2] = issue(c + 1)
            slot = c % 2
            for cp in inflight[slot]:
                cp.wait()

            def row_body(row, tot, slot=slot):
                acc0 = jnp.zeros((L,), jnp.float32)
                acc1 = jnp.zeros((L,), jnp.float32)
                for k in range(KD):
                    h = hb[slot, row, pl.ds(k * L, L)]
                    r = rb[slot, row, pl.ds(k * L, L)]
                    t = tb[slot, row, pl.ds(k * L, L)]
                    d = h + r - t
                    if k % 2 == 0:
                        acc0 = acc0 + d * d
                    else:
                        acc1 = acc1 + d * d
                s = jnp.sum(acc0 + acc1)
                return jnp.where(lane == lax.rem(row, L), s, tot)

            def group(g, _, c=c, slot=slot):
                tot = lax.fori_loop(
                    g * L, (g + 1) * L, row_body, jnp.zeros((L,), jnp.float32))
                outb[pl.ds(c * CH + g * L, L)] = _fast_sqrt(tot)
                return 0

            lax.fori_loop(0, CH // L, group, 0)

        pltpu.sync_copy(outb, out_h.at[wid])

    return sc_kernel


def kernel(head, relation, tail, entity_emb, relation_emb):
    B = head.shape[0]
    D = entity_emb.shape[1]
    BPW = B // NW
    CH = 128
    NCH = BPW // CH
    hrt = jnp.stack([head.reshape(NW, NCH, CH),
                     relation.reshape(NW, NCH, CH),
                     tail.reshape(NW, NCH, CH)], axis=1)
    sc_kernel = _build_sc_kernel(B, D)
    score = sc_kernel(hrt, entity_emb, relation_emb)
    return score.reshape(B)


# speedup vs baseline: 2.6754x; 1.0542x over previous
import functools

import jax
import jax.numpy as jnp
from jax import lax
from jax.experimental import pallas as pl
from jax.experimental.pallas import tpu as pltpu
from jax.experimental.pallas import tpu_sc as plsc

NC = 2
NS = 16
L = 16
NW = NC * NS


def _fast_sqrt(x):
    x = jnp.maximum(x, jnp.float32(1e-30))
    i = lax.bitcast_convert_type(x, jnp.int32)
    i = jnp.int32(0x5F3759DF) - lax.shift_right_arithmetic(i, jnp.int32(1))
    y = lax.bitcast_convert_type(i, jnp.float32)
    y = y * (jnp.float32(1.5) - jnp.float32(0.5) * x * y * y)
    y = y * (jnp.float32(1.5) - jnp.float32(0.5) * x * y * y)
    return x * y


@functools.lru_cache(maxsize=None)
def _build_sc_kernel(B, D):
    BPW = B // NW
    CH = 128
    NCH = BPW // CH
    KD = D // L

    mesh = plsc.VectorSubcoreMesh(core_axis_name="c", subcore_axis_name="s")

    @functools.partial(
        pl.kernel,
        mesh=mesh,
        compiler_params=pltpu.CompilerParams(needs_layout_passes=False),
        out_type=jax.ShapeDtypeStruct((B,), jnp.float32),
        scratch_types=[
            pltpu.VMEM((NCH, CH), jnp.int32),
            pltpu.VMEM((NCH, CH), jnp.int32),
            pltpu.VMEM((NCH, CH), jnp.int32),
            pltpu.VMEM((2, CH, D), jnp.float32),
            pltpu.VMEM((2, CH, D), jnp.float32),
            pltpu.VMEM((2, CH, D), jnp.float32),
            pltpu.VMEM((BPW,), jnp.float32),
            pltpu.SemaphoreType.DMA,
            pltpu.SemaphoreType.DMA,
            pltpu.SemaphoreType.DMA,
            pltpu.SemaphoreType.DMA,
            pltpu.SemaphoreType.DMA,
            pltpu.SemaphoreType.DMA,
        ],
    )
    def sc_kernel(head_h, rel_h, tail_h, ent_h, remb_h, out_h,
                  idx_hh, idx_rr, idx_tt, hb, rb, tb, outb,
                  sh0, sh1, sr0, sr1, st0, st1):
        wid = lax.axis_index("s") * NC + lax.axis_index("c")
        cpi_h = pltpu.async_copy(head_h.at[wid], idx_hh, sh0)
        cpi_r = pltpu.async_copy(rel_h.at[wid], idx_rr, sr0)
        cpi_t = pltpu.async_copy(tail_h.at[wid], idx_tt, st0)
        cpi_h.wait()
        cpi_r.wait()
        cpi_t.wait()

        sems = ((sh0, sr0, st0), (sh1, sr1, st1))
        lane = lax.iota(jnp.int32, L)

        def issue(c):
            slot = c % 2
            sh, sr, st = sems[slot]
            return (
                pltpu.async_copy(ent_h.at[idx_hh.at[c]], hb.at[slot], sh),
                pltpu.async_copy(remb_h.at[idx_rr.at[c]], rb.at[slot], sr),
                pltpu.async_copy(ent_h.at[idx_tt.at[c]], tb.at[slot], st),
            )

        inflight = [None, None]
        inflight[0] = issue(0)
        for c in range(NCH):
            if c + 1 < NCH:
                inflight[(c + 1) % 2] = issue(c + 1)
            slot = c % 2
            for cp in inflight[slot]:
                cp.wait()

            def row_body(row, tot, slot=slot):
                acc0 = jnp.zeros((L,), jnp.float32)
                acc1 = jnp.zeros((L,), jnp.float32)
                for k in range(KD):
                    h = hb[slot, row, pl.ds(k * L, L)]
                    r = rb[slot, row, pl.ds(k * L, L)]
                    t = tb[slot, row, pl.ds(k * L, L)]
                    d = h + r - t
                    if k % 2 == 0:
                        acc0 = acc0 + d * d
                    else:
                        acc1 = acc1 + d * d
                s = jnp.sum(acc0 + acc1)
                return jnp.where(lane == lax.rem(row, L), s, tot)

            def group(g, _, c=c, slot=slot):
                tot = lax.fori_loop(
                    g * L, (g + 1) * L, row_body, jnp.zeros((L,), jnp.float32))
                outb[pl.ds(c * CH + g * L, L)] = _fast_sqrt(tot)
                return 0

            lax.fori_loop(0, CH // L, group, 0)

        pltpu.sync_copy(outb, out_h.at[pl.ds(wid * BPW, BPW)])

    return sc_kernel


def kernel(head, relation, tail, entity_emb, relation_emb):
    B = head.shape[0]
    D = entity_emb.shape[1]
    BPW = B // NW
    CH = 128
    NCH = BPW // CH
    sc_kernel = _build_sc_kernel(B, D)
    return sc_kernel(
        head.reshape(NW, NCH, CH),
        relation.reshape(NW, NCH, CH),
        tail.reshape(NW, NCH, CH),
        entity_emb,
        relation_emb,
    )
